# baseline (speedup 1.0000x reference)
import functools

import jax
import jax.numpy as jnp
from jax import lax
from jax.experimental import pallas as pl
from jax.experimental.pallas import tpu as pltpu
from jax.experimental.pallas import tpu_sc as plsc

N = 50000
C = 128
K = 9
M_NEW = 65536
OUT_ROWS = M_NEW // 4

NW = 32
NP = 50176
SB = 392
NCH = NP // SB
CH_A = 5
CH_B = 3
NCH_A = 16 * CH_A
ZROW = NP
TAILZ = 512
BX = 256
SB5 = 64
OPW = OUT_ROWS // NW

_MESH = plsc.VectorSubcoreMesh(core_axis_name="c", subcore_axis_name="s")


def _mm_body(relu_input):
    def body(x_ref, w_ref, b_ref, o_ref):
        xb = x_ref[...]
        if relu_input:
            xb = jnp.maximum(xb, 0).astype(jnp.bfloat16)
        res = jnp.dot(xb, w_ref[...], preferred_element_type=jnp.float32)
        o_ref[...] = res + b_ref[...]
    return body


def _mm_call(xin, wc, b9, relu_input):
    return pl.pallas_call(
        _mm_body(relu_input),
        grid=(NP // BX,),
        in_specs=[
            pl.BlockSpec((BX, C), lambda i: (i, 0)),
            pl.BlockSpec((C, K * C), lambda i: (0, 0)),
            pl.BlockSpec((1, K * C), lambda i: (0, 0)),
        ],
        out_specs=pl.BlockSpec((BX, K * C), lambda i: (i, 0)),
        out_shape=jax.ShapeDtypeStruct((NP, K * C), jnp.float32),
    )(xin, wc, b9)


def _make_sc_conv(zero_tail):
    out_rows = NP + (TAILZ if zero_tail else 0)

    @functools.partial(
        pl.kernel,
        mesh=_MESH,
        out_type=jax.ShapeDtypeStruct((out_rows, C), jnp.float32),
        scratch_types=[
            pltpu.VMEM((K * SB,), jnp.int32),
            pltpu.VMEM((SB, C), jnp.float32),
            pltpu.SemaphoreType.DMA,
        ],
    )
    def sc_conv(table, idx3, out, idx_v, acc_v, sem):
        core = lax.axis_index("c")
        sub_ix = lax.axis_index("s")
        cnt = jnp.where(core == 0, CH_A, CH_B)
        chunk0 = jnp.where(core == 0, sub_ix * CH_A, NCH_A + sub_ix * CH_B)
        z = jnp.zeros((16,), jnp.float32)

        def sub(s, carry):
            chunk = chunk0 + s
            row0 = chunk * SB
            pltpu.sync_copy(idx3.at[chunk], idx_v)

            def zrow(r, c2):
                for t in range(C // 16):
                    acc_v[r, pl.ds(t * 16, 16)] = z
                return c2

            lax.fori_loop(0, SB, zrow, 0)
            descs = [
                pltpu.async_copy(
                    table.at[idx_v.at[pl.ds(k * SB, SB)]], acc_v, sem,
                    add=True)
                for k in range(K)
            ]
            for d in descs:
                d.wait()
            pltpu.sync_copy(acc_v, out.at[pl.ds(row0, SB)])
            return carry

        lax.fori_loop(0, cnt, sub, 0)

        if zero_tail:
            wid = sub_ix * 2 + core

            def ztrow(r, c2):
                for t in range(C // 16):
                    acc_v[r, pl.ds(t * 16, 16)] = z
                return c2

            lax.fori_loop(0, 16, ztrow, 0)
            pltpu.sync_copy(acc_v.at[pl.ds(0, 16)],
                            out.at[pl.ds(NP + wid * 16, 16)])

    return sc_conv


_sc_conv_plain = _make_sc_conv(zero_tail=False)
_sc_conv_tail = _make_sc_conv(zero_tail=True)


@functools.partial(
    pl.kernel,
    mesh=_MESH,
    out_type=jax.ShapeDtypeStruct((OUT_ROWS, C), jnp.float32),
    scratch_types=[
        pltpu.VMEM((4 * SB5,), jnp.int32),
        pltpu.VMEM((4 * SB5, C), jnp.float32),
        pltpu.VMEM((SB5, C), jnp.float32),
        pltpu.SemaphoreType.DMA,
    ],
)
def _sc_pool(table, perm_h, out, pidx_v, gbuf_v, obuf_v, sem):
    wid = lax.axis_index("s") * 2 + lax.axis_index("c")
    iota16 = lax.iota(jnp.int32, 16)

    def sub(s, carry):
        ob = wid * OPW + s * SB5
        pltpu.sync_copy(perm_h.at[pl.ds(4 * ob, 4 * SB5)], pidx_v)

        def fix(j, c2):
            sl = pl.ds(j * 16, 16)
            v = pidx_v[sl]
            zr = ZROW + ((j * 16 + iota16) & (TAILZ - 1))
            pidx_v[sl] = jnp.where(v < N, v, zr)
            return c2

        lax.fori_loop(0, (4 * SB5) // 16, fix, 0)
        pltpu.async_copy(table.at[pidx_v], gbuf_v, sem).wait()

        def pool(j, c2):
            for t in range(C // 16):
                sl = pl.ds(t * 16, 16)
                m01 = jnp.maximum(gbuf_v[4 * j, sl], gbuf_v[4 * j + 1, sl])
                m23 = jnp.maximum(gbuf_v[4 * j + 2, sl], gbuf_v[4 * j + 3, sl])
                obuf_v[j, sl] = jnp.maximum(jnp.maximum(m01, m23), 0.0)
            return c2

        lax.fori_loop(0, SB5, pool, 0)
        pltpu.sync_copy(obuf_v, out.at[pl.ds(ob, SB5)])
        return carry

    lax.fori_loop(0, OPW // SB5, sub, 0)


def kernel(x, adj, perm, W1, b1, W2, b2):
    xp = jnp.pad(x, ((0, NP - N), (0, 0))).astype(jnp.bfloat16)
    adjp = jnp.pad(adj, ((0, NP - N), (0, 0))).astype(jnp.int32)
    idx3 = (adjp.reshape(NCH, SB, K) * K
            + jnp.arange(K, dtype=jnp.int32)[None, None, :])
    idx3 = idx3.transpose(0, 2, 1).reshape(NCH, K * SB)
    wc1 = W1.reshape(K, C, C).transpose(1, 0, 2).reshape(C, K * C).astype(
        jnp.bfloat16)
    wc2 = W2.reshape(K, C, C).transpose(1, 0, 2).reshape(C, K * C).astype(
        jnp.bfloat16)
    b19 = jnp.tile(b1 / K, K).reshape(1, K * C)
    b29 = jnp.tile(b2 / K, K).reshape(1, K * C)

    y1 = _mm_call(xp, wc1, b19, relu_input=False).reshape(NP * K, C)
    net1 = _sc_conv_plain(y1, idx3)
    y2 = _mm_call(net1, wc2, b29, relu_input=True).reshape(NP * K, C)
    net2 = _sc_conv_tail(y2, idx3)
    return _sc_pool(net2, perm)


# speedup vs baseline: 1.5816x; 1.5816x over previous
import functools

import jax
import jax.numpy as jnp
from jax import lax
from jax.experimental import pallas as pl
from jax.experimental.pallas import tpu as pltpu
from jax.experimental.pallas import tpu_sc as plsc

N = 50000
C = 128
K = 9
M_NEW = 65536
OUT_ROWS = M_NEW // 4

NW = 32
NP = 50176
SB = 392
NCH = NP // SB
CH_A = 5
CH_B = 3
NCH_A = 16 * CH_A
ZROW = NP
TAILZ = 512
BX = 256
SB5 = 64
OPW = OUT_ROWS // NW

_MESH = plsc.VectorSubcoreMesh(core_axis_name="c", subcore_axis_name="s")


def _mm_body(relu_input):
    def body(x_ref, w_ref, b_ref, o_ref):
        xb = x_ref[...]
        if relu_input:
            xb = jnp.maximum(xb, 0).astype(jnp.bfloat16)
        res = jnp.dot(xb, w_ref[...], preferred_element_type=jnp.float32)
        for k in range(K):
            o_ref[k] = res[:, k * C:(k + 1) * C] + b_ref[...]
    return body


def _mm_call(xin, wc, b9, relu_input):
    return pl.pallas_call(
        _mm_body(relu_input),
        grid=(NP // BX,),
        in_specs=[
            pl.BlockSpec((BX, C), lambda i: (i, 0)),
            pl.BlockSpec((C, K * C), lambda i: (0, 0)),
            pl.BlockSpec((1, C), lambda i: (0, 0)),
        ],
        out_specs=pl.BlockSpec((K, BX, C), lambda i: (0, i, 0)),
        out_shape=jax.ShapeDtypeStruct((K, NP, C), jnp.float32),
    )(xin, wc, b9)


def _make_sc_conv(zero_tail):
    out_rows = NP + (TAILZ if zero_tail else 0)

    @functools.partial(
        pl.kernel,
        mesh=_MESH,
        out_type=jax.ShapeDtypeStruct((out_rows, C), jnp.float32),
        scratch_types=[
            pltpu.VMEM((K * SB,), jnp.int32),
            pltpu.VMEM((SB, C), jnp.float32),
            pltpu.SemaphoreType.DMA,
        ],
    )
    def sc_conv(table, idx3, out, idx_v, acc_v, sem):
        core = lax.axis_index("c")
        sub_ix = lax.axis_index("s")
        cnt = jnp.where(core == 0, CH_A, CH_B)
        chunk0 = jnp.where(core == 0, sub_ix * CH_A, NCH_A + sub_ix * CH_B)
        z = jnp.zeros((16,), jnp.float32)

        def sub(s, carry):
            chunk = chunk0 + s
            row0 = chunk * SB
            pltpu.sync_copy(idx3.at[chunk], idx_v)

            def zrow(r, c2):
                for t in range(C // 16):
                    acc_v[r, pl.ds(t * 16, 16)] = z
                return c2

            lax.fori_loop(0, SB, zrow, 0)
            descs = [
                pltpu.async_copy(
                    table.at[idx_v.at[pl.ds(k * SB, SB)]], acc_v, sem,
                    add=True)
                for k in range(K)
            ]
            for d in descs:
                d.wait()
            pltpu.sync_copy(acc_v, out.at[pl.ds(row0, SB)])
            return carry

        lax.fori_loop(0, cnt, sub, 0)

        if zero_tail:
            wid = sub_ix * 2 + core

            def ztrow(r, c2):
                for t in range(C // 16):
                    acc_v[r, pl.ds(t * 16, 16)] = z
                return c2

            lax.fori_loop(0, 16, ztrow, 0)
            pltpu.sync_copy(acc_v.at[pl.ds(0, 16)],
                            out.at[pl.ds(NP + wid * 16, 16)])

    return sc_conv


_sc_conv_plain = _make_sc_conv(zero_tail=False)
_sc_conv_tail = _make_sc_conv(zero_tail=True)


@functools.partial(
    pl.kernel,
    mesh=_MESH,
    out_type=jax.ShapeDtypeStruct((OUT_ROWS, C), jnp.float32),
    scratch_types=[
        pltpu.VMEM((4 * SB5,), jnp.int32),
        pltpu.VMEM((4 * SB5, C), jnp.float32),
        pltpu.VMEM((SB5, C), jnp.float32),
        pltpu.SemaphoreType.DMA,
    ],
)
def _sc_pool(table, perm_h, out, pidx_v, gbuf_v, obuf_v, sem):
    wid = lax.axis_index("s") * 2 + lax.axis_index("c")
    iota16 = lax.iota(jnp.int32, 16)

    def sub(s, carry):
        ob = wid * OPW + s * SB5
        pltpu.sync_copy(perm_h.at[pl.ds(4 * ob, 4 * SB5)], pidx_v)

        def fix(j, c2):
            sl = pl.ds(j * 16, 16)
            v = pidx_v[sl]
            zr = ZROW + ((j * 16 + iota16) & (TAILZ - 1))
            pidx_v[sl] = jnp.where(v < N, v, zr)
            return c2

        lax.fori_loop(0, (4 * SB5) // 16, fix, 0)
        pltpu.async_copy(table.at[pidx_v], gbuf_v, sem).wait()

        def pool(j, c2):
            for t in range(C // 16):
                sl = pl.ds(t * 16, 16)
                m01 = jnp.maximum(gbuf_v[4 * j, sl], gbuf_v[4 * j + 1, sl])
                m23 = jnp.maximum(gbuf_v[4 * j + 2, sl], gbuf_v[4 * j + 3, sl])
                obuf_v[j, sl] = jnp.maximum(jnp.maximum(m01, m23), 0.0)
            return c2

        lax.fori_loop(0, SB5, pool, 0)
        pltpu.sync_copy(obuf_v, out.at[pl.ds(ob, SB5)])
        return carry

    lax.fori_loop(0, OPW // SB5, sub, 0)


def kernel(x, adj, perm, W1, b1, W2, b2):
    xp = jnp.pad(x, ((0, NP - N), (0, 0))).astype(jnp.bfloat16)
    adjp = jnp.pad(adj, ((0, NP - N), (0, 0))).astype(jnp.int32)
    idx3 = (adjp.reshape(NCH, SB, K)
            + (jnp.arange(K, dtype=jnp.int32) * NP)[None, None, :])
    idx3 = idx3.transpose(0, 2, 1).reshape(NCH, K * SB)
    wc1 = W1.reshape(K, C, C).transpose(1, 0, 2).reshape(C, K * C).astype(
        jnp.bfloat16)
    wc2 = W2.reshape(K, C, C).transpose(1, 0, 2).reshape(C, K * C).astype(
        jnp.bfloat16)
    b19 = (b1 / K).reshape(1, C)
    b29 = (b2 / K).reshape(1, C)

    y1 = _mm_call(xp, wc1, b19, relu_input=False).reshape(K * NP, C)
    net1 = _sc_conv_plain(y1, idx3)
    y2 = _mm_call(net1, wc2, b29, relu_input=True).reshape(K * NP, C)
    net2 = _sc_conv_tail(y2, idx3)
    return _sc_pool(net2, perm)


# speedup vs baseline: 1.8437x; 1.1657x over previous
import functools

import jax
import jax.numpy as jnp
from jax import lax
from jax.experimental import pallas as pl
from jax.experimental.pallas import tpu as pltpu
from jax.experimental.pallas import tpu_sc as plsc

N = 50000
C = 128
K = 9
M_NEW = 65536
OUT_ROWS = M_NEW // 4

NW = 32
NP = 50176
SB = 392
NCH = NP // SB
CH_A = 5
CH_B = 3
NCH_A = 16 * CH_A
ZROW = NP
TAILZ = 512
BX = 512
SB5 = 64
OPW = OUT_ROWS // NW

_MESH = plsc.VectorSubcoreMesh(core_axis_name="c", subcore_axis_name="s")


def _mm_body(relu_input):
    def body(x_ref, w_ref, b_ref, o_ref):
        xb = x_ref[...]
        if relu_input:
            xb = jnp.maximum(xb, 0)
        xb = xb.astype(jnp.bfloat16)
        res = jnp.dot(xb, w_ref[...], preferred_element_type=jnp.float32)
        for k in range(K):
            o_ref[k] = res[:, k * C:(k + 1) * C] + b_ref[...]
    return body


def _mm_call(xin, wc, b9, relu_input):
    return pl.pallas_call(
        _mm_body(relu_input),
        grid=(NP // BX,),
        in_specs=[
            pl.BlockSpec((BX, C), lambda i: (i, 0)),
            pl.BlockSpec((C, K * C), lambda i: (0, 0)),
            pl.BlockSpec((1, C), lambda i: (0, 0)),
        ],
        out_specs=pl.BlockSpec((K, BX, C), lambda i: (0, i, 0)),
        out_shape=jax.ShapeDtypeStruct((K, NP, C), jnp.float32),
    )(xin, wc, b9)


def _make_sc_conv(zero_tail):
    out_rows = NP + (TAILZ if zero_tail else 0)

    @functools.partial(
        pl.kernel,
        mesh=_MESH,
        out_type=jax.ShapeDtypeStruct((out_rows, C), jnp.float32),
        scratch_types=[
            pltpu.VMEM((K * SB,), jnp.int32),
            pltpu.VMEM((SB, C), jnp.float32),
            pltpu.SemaphoreType.DMA,
        ],
    )
    def sc_conv(table, idx3, out, idx_v, acc_v, sem):
        core = lax.axis_index("c")
        sub_ix = lax.axis_index("s")
        cnt = jnp.where(core == 0, CH_A, CH_B)
        chunk0 = jnp.where(core == 0, sub_ix * CH_A, NCH_A + sub_ix * CH_B)
        z = jnp.zeros((16,), jnp.float32)

        def sub(s, carry):
            chunk = chunk0 + s
            row0 = chunk * SB
            pltpu.sync_copy(idx3.at[chunk], idx_v)

            def zrow(r, c2):
                for t in range(C // 16):
                    acc_v[r, pl.ds(t * 16, 16)] = z
                return c2

            lax.fori_loop(0, SB, zrow, 0)
            descs = [
                pltpu.async_copy(
                    table.at[idx_v.at[pl.ds(k * SB, SB)]], acc_v, sem,
                    add=True)
                for k in range(K)
            ]
            for d in descs:
                d.wait()
            pltpu.sync_copy(acc_v, out.at[pl.ds(row0, SB)])
            return carry

        lax.fori_loop(0, cnt, sub, 0)

        if zero_tail:
            wid = sub_ix * 2 + core

            def ztrow(r, c2):
                for t in range(C // 16):
                    acc_v[r, pl.ds(t * 16, 16)] = z
                return c2

            lax.fori_loop(0, 16, ztrow, 0)
            pltpu.sync_copy(acc_v.at[pl.ds(0, 16)],
                            out.at[pl.ds(NP + wid * 16, 16)])

    return sc_conv


_sc_conv_plain = _make_sc_conv(zero_tail=False)
_sc_conv_tail = _make_sc_conv(zero_tail=True)


@functools.partial(
    pl.kernel,
    mesh=_MESH,
    out_type=jax.ShapeDtypeStruct((OUT_ROWS, C), jnp.float32),
    scratch_types=[
        pltpu.VMEM((4 * SB5,), jnp.int32),
        pltpu.VMEM((4 * SB5, C), jnp.float32),
        pltpu.VMEM((SB5, C), jnp.float32),
        pltpu.SemaphoreType.DMA,
    ],
)
def _sc_pool(table, perm_h, out, pidx_v, gbuf_v, obuf_v, sem):
    wid = lax.axis_index("s") * 2 + lax.axis_index("c")
    iota16 = lax.iota(jnp.int32, 16)

    def sub(s, carry):
        ob = wid * OPW + s * SB5
        pltpu.sync_copy(perm_h.at[pl.ds(4 * ob, 4 * SB5)], pidx_v)

        def fix(j, c2):
            sl = pl.ds(j * 16, 16)
            v = pidx_v[sl]
            zr = ZROW + ((j * 16 + iota16) & (TAILZ - 1))
            pidx_v[sl] = jnp.where(v < N, v, zr)
            return c2

        lax.fori_loop(0, (4 * SB5) // 16, fix, 0)
        pltpu.async_copy(table.at[pidx_v], gbuf_v, sem).wait()

        def pool(j, c2):
            for t in range(C // 16):
                sl = pl.ds(t * 16, 16)
                m01 = jnp.maximum(gbuf_v[4 * j, sl], gbuf_v[4 * j + 1, sl])
                m23 = jnp.maximum(gbuf_v[4 * j + 2, sl], gbuf_v[4 * j + 3, sl])
                obuf_v[j, sl] = jnp.maximum(jnp.maximum(m01, m23), 0.0)
            return c2

        lax.fori_loop(0, SB5, pool, 0)
        pltpu.sync_copy(obuf_v, out.at[pl.ds(ob, SB5)])
        return carry

    lax.fori_loop(0, OPW // SB5, sub, 0)


def kernel(x, adj, perm, W1, b1, W2, b2):
    xp = jnp.pad(x, ((0, NP - N), (0, 0)))
    adjp = jnp.pad(adj, ((0, NP - N), (0, 0))).astype(jnp.int32)
    idx3 = (adjp.reshape(NCH, SB, K)
            + (jnp.arange(K, dtype=jnp.int32) * NP)[None, None, :])
    idx3 = idx3.transpose(0, 2, 1).reshape(NCH, K * SB)
    wc1 = W1.reshape(K, C, C).transpose(1, 0, 2).reshape(C, K * C).astype(
        jnp.bfloat16)
    wc2 = W2.reshape(K, C, C).transpose(1, 0, 2).reshape(C, K * C).astype(
        jnp.bfloat16)
    b19 = (b1 / K).reshape(1, C)
    b29 = (b2 / K).reshape(1, C)

    y1 = _mm_call(xp, wc1, b19, relu_input=False).reshape(K * NP, C)
    net1 = _sc_conv_plain(y1, idx3)
    y2 = _mm_call(net1, wc2, b29, relu_input=True).reshape(K * NP, C)
    net2 = _sc_conv_tail(y2, idx3)
    return _sc_pool(net2, perm)
